# baseline (speedup 1.0000x reference)
import functools

import jax
import jax.numpy as jnp
from jax import lax
from jax.experimental import pallas as pl
from jax.experimental.pallas import tpu as pltpu
from jax.experimental.pallas import tpu_sc as plsc

NC = 2
NS = 16
NW = NC * NS
K = 128
KG = 64
B = 4


SLOW_C = 1
SBLK = 32


def _sc_aggregate(h, src2, dst2, zeros2d, npad, cw_s, cw_f):
    n_chunks_total, _ = src2.shape
    D = h.shape[1]
    rows_per_tile = npad // NS
    tbh = SBLK // B

    mesh = plsc.VectorSubcoreMesh(core_axis_name="c", subcore_axis_name="s")

    @functools.partial(
        pl.kernel,
        out_type=jax.ShapeDtypeStruct((NC, npad, D), jnp.float32),
        mesh=mesh,
        scratch_types=[
            pltpu.VMEM((SBLK, KG), jnp.int32),
            pltpu.VMEM((SBLK, KG), jnp.int32),
        ] + [pltpu.VMEM((KG, D), jnp.float32)] * B + [
            pltpu.VMEM_SHARED((npad, D), jnp.float32),
        ] + [pltpu.SemaphoreType.DMA] * (2 * B),
    )
    def agg(h_hbm, src_hbm, dst_hbm, z_hbm, out_hbm,
            srcs_v, dsts_v, *rest):
        rows, acc_sh, sems = rest[:B], rest[B], rest[B + 1:]
        gs, ss = sems[:B], sems[B:]
        c = lax.axis_index("c")
        s = lax.axis_index("s")
        row0 = s * rows_per_tile

        def gwait(b, j):
            pltpu.make_async_copy(h_hbm.at[srcs_v.at[j]],
                                  rows[b], gs[b]).wait()

        def swait(b, j):
            pltpu.make_async_copy(rows[b],
                                  acc_sh.at[dsts_v.at[j]], ss[b]).wait()

        def step(t, refill):
            jb = t * B
            for b in range(B):
                gwait(b, jb + b)
                pltpu.async_copy(rows[b], acc_sh.at[dsts_v.at[jb + b]],
                                 ss[b], add=True)
            for b in range(B):
                swait(b, jb + b)
                if refill:
                    pltpu.async_copy(h_hbm.at[srcs_v.at[jb + B + b]],
                                     rows[b], gs[b])

        def body(t, carry):
            step(t, refill=True)
            return carry

        def run(cw, cb0):
            for blk in range(cw // SBLK):
                base = cb0 + blk * SBLK
                pltpu.sync_copy(src_hbm.at[pl.ds(base, SBLK)], srcs_v)
                pltpu.sync_copy(dst_hbm.at[pl.ds(base, SBLK)], dsts_v)
                for b in range(B):
                    pltpu.async_copy(h_hbm.at[srcs_v.at[b]], rows[b], gs[b])
                lax.fori_loop(0, tbh - 1, body, 0)
                step(tbh - 1, refill=False)

        pltpu.sync_copy(z_hbm.at[pl.ds(row0, rows_per_tile)],
                        acc_sh.at[pl.ds(row0, rows_per_tile)])
        plsc.subcore_barrier()

        @pl.when(c == SLOW_C)
        def _():
            run(cw_s, s * cw_s)

        @pl.when(c != SLOW_C)
        def _():
            run(cw_f, NS * cw_s + s * cw_f)

        plsc.subcore_barrier()
        pltpu.sync_copy(acc_sh.at[pl.ds(row0, rows_per_tile)],
                        out_hbm.at[c, pl.ds(row0, rows_per_tile)])

    return agg(h, src2, dst2, zeros2d)


def _sc_count(dst2, zeros2d, ones2d, npad):
    n_chunks_total, _ = dst2.shape
    D = zeros2d.shape[1]
    chunks_per_w = n_chunks_total // NW
    rows_per_tile = npad // NS
    SB = 8
    ng = chunks_per_w // SB

    mesh = plsc.VectorSubcoreMesh(core_axis_name="c", subcore_axis_name="s")

    @functools.partial(
        pl.kernel,
        out_type=jax.ShapeDtypeStruct((NC, npad, D), jnp.float32),
        mesh=mesh,
        scratch_types=[
            pltpu.VMEM((chunks_per_w, K), jnp.int32),
            pltpu.VMEM((K, D), jnp.float32),
            pltpu.VMEM_SHARED((npad, D), jnp.float32),
        ] + [pltpu.SemaphoreType.DMA] * SB,
    )
    def cnt(dst_hbm, z_hbm, ones_hbm, out_hbm, dsts_v, ones_v, acc_sh, *sems):
        c = lax.axis_index("c")
        s = lax.axis_index("s")
        row0 = s * rows_per_tile
        wid = c * NS + s
        cb0 = wid * chunks_per_w
        pltpu.sync_copy(dst_hbm.at[pl.ds(cb0, chunks_per_w)], dsts_v)
        pltpu.sync_copy(ones_hbm.at[pl.ds(0, K)], ones_v)
        pltpu.sync_copy(z_hbm.at[pl.ds(row0, rows_per_tile)],
                        acc_sh.at[pl.ds(row0, rows_per_tile)])
        plsc.subcore_barrier()

        for b in range(SB):
            pltpu.async_copy(ones_v, acc_sh.at[dsts_v.at[b]], sems[b],
                             add=True)

        def body(t, carry):
            jb = t * SB
            for b in range(SB):
                pltpu.make_async_copy(ones_v, acc_sh.at[dsts_v.at[jb - SB + b]],
                                      sems[b]).wait()
                pltpu.async_copy(ones_v, acc_sh.at[dsts_v.at[jb + b]],
                                 sems[b], add=True)
            return carry

        lax.fori_loop(1, ng, body, 0)
        for b in range(SB):
            pltpu.make_async_copy(ones_v,
                                  acc_sh.at[dsts_v.at[(ng - 1) * SB + b]],
                                  sems[b]).wait()
        plsc.subcore_barrier()
        pltpu.sync_copy(acc_sh.at[pl.ds(row0, rows_per_tile)],
                        out_hbm.at[c, pl.ds(row0, rows_per_tile)])

    return cnt(dst2, zeros2d, ones2d)


def _tc_combine(cntp, part, h, Wl, Wr, bl2, relu):
    npad, D = h.shape
    R = 1280
    grid = npad // R

    def body(cnt_ref, part_ref, h_ref, wl_ref, wr_ref, bl_ref, out_ref):
        cb = cnt_ref[0] + cnt_ref[1]
        inv = 1.0 / jnp.maximum(cb, 1.0)
        mean = (part_ref[0] + part_ref[1]) * inv
        acc = lax.dot_general(mean, wl_ref[...], (((1,), (1,)), ((), ())),
                              preferred_element_type=jnp.float32)
        acc = acc + lax.dot_general(h_ref[...], wr_ref[...],
                                    (((1,), (1,)), ((), ())),
                                    preferred_element_type=jnp.float32)
        acc = acc + bl_ref[...]
        if relu:
            acc = jnp.maximum(acc, 0.0)
        out_ref[...] = acc

    return pl.pallas_call(
        body,
        grid=(grid,),
        in_specs=[
            pl.BlockSpec((NC, R, D), lambda i: (0, i, 0)),
            pl.BlockSpec((NC, R, D), lambda i: (0, i, 0)),
            pl.BlockSpec((R, D), lambda i: (i, 0)),
            pl.BlockSpec((D, D), lambda i: (0, 0)),
            pl.BlockSpec((D, D), lambda i: (0, 0)),
            pl.BlockSpec((1, D), lambda i: (0, 0)),
        ],
        out_specs=pl.BlockSpec((R, D), lambda i: (i, 0)),
        out_shape=jax.ShapeDtypeStruct((npad, D), jnp.float32),
    )(cntp, part, h, Wl, Wr, bl2)


def kernel(x, edge_index, Wl0, bl0, Wr0, Wl1, bl1, Wr1, Wl2, bl2, Wr2):
    N, D = x.shape
    E = edge_index.shape[1]
    npad = ((N + 1279) // 1280) * 1280
    echunk = NW * K * 8
    epad = ((E + echunk - 1) // echunk) * echunk

    src = edge_index[0]
    dst = edge_index[1]
    pad = epad - E
    srcp = jnp.concatenate([src, jnp.zeros((pad,), jnp.int32)])
    dstp = jnp.concatenate([dst, jnp.full((pad,), N, jnp.int32)])
    src4 = srcp.reshape(-1, KG)
    dst4 = dstp.reshape(-1, KG)
    dst2 = dstp.reshape(-1, K)

    zeros2d = jnp.zeros((npad, D), jnp.float32)
    ones2d = jnp.ones((npad, D), jnp.float32)
    xp = jnp.concatenate([x, jnp.zeros((npad - N, D), jnp.float32)], axis=0)

    cntp = _sc_count(dst2, zeros2d, ones2d, npad)

    n_chunks_total = epad // KG
    tot_per_ns = n_chunks_total // NS
    cw_s = max(SBLK,
               int(round(n_chunks_total * 0.2 / (NS * SBLK))) * SBLK)
    cw_f = tot_per_ns - cw_s

    h = xp
    for Wl, bl, Wr, relu in ((Wl0, bl0, Wr0, True),
                             (Wl1, bl1, Wr1, True),
                             (Wl2, bl2, Wr2, False)):
        part = _sc_aggregate(h, src4, dst4, zeros2d, npad, cw_s, cw_f)
        h = _tc_combine(cntp, part, h, Wl, Wr, bl.reshape(1, D), relu)
    return h[:N]


# speedup vs baseline: 1.0753x; 1.0753x over previous
import functools

import jax
import jax.numpy as jnp
from jax import lax
from jax.experimental import pallas as pl
from jax.experimental.pallas import tpu as pltpu
from jax.experimental.pallas import tpu_sc as plsc

NC = 2
NS = 16
NW = NC * NS
K = 128
KG = 64
B = 4


SLOW_C = 1
SBLK = 32


def _sc_aggregate(h, src2, dst2, zeros2d, npad, cw_s, cw_f):
    n_chunks_total, _ = src2.shape
    D = h.shape[1]
    rows_per_tile = npad // NS
    tbh = SBLK // B

    mesh = plsc.VectorSubcoreMesh(core_axis_name="c", subcore_axis_name="s")

    @functools.partial(
        pl.kernel,
        out_type=jax.ShapeDtypeStruct((NC, npad, D), jnp.float32),
        mesh=mesh,
        scratch_types=[
            pltpu.VMEM((SBLK, KG), jnp.int32),
            pltpu.VMEM((SBLK, KG), jnp.int32),
        ] + [pltpu.VMEM((KG, D), jnp.float32)] * B + [
            pltpu.VMEM_SHARED((npad, D), jnp.float32),
        ] + [pltpu.SemaphoreType.DMA] * (2 * B),
    )
    def agg(h_hbm, src_hbm, dst_hbm, z_hbm, out_hbm,
            srcs_v, dsts_v, *rest):
        rows, acc_sh, sems = rest[:B], rest[B], rest[B + 1:]
        gs, ss = sems[:B], sems[B:]
        c = lax.axis_index("c")
        s = lax.axis_index("s")
        row0 = s * rows_per_tile

        def gwait(b, j):
            pltpu.make_async_copy(h_hbm.at[srcs_v.at[j]],
                                  rows[b], gs[b]).wait()

        def swait(b, j):
            pltpu.make_async_copy(rows[b],
                                  acc_sh.at[dsts_v.at[j]], ss[b]).wait()

        def step(t, refill):
            jb = t * B
            for b in range(B):
                gwait(b, jb + b)
                pltpu.async_copy(rows[b], acc_sh.at[dsts_v.at[jb + b]],
                                 ss[b], add=True)
            for b in range(B):
                swait(b, jb + b)
                if refill:
                    pltpu.async_copy(h_hbm.at[srcs_v.at[jb + B + b]],
                                     rows[b], gs[b])

        def body(t, carry):
            step(t, refill=True)
            return carry

        def run(cw, cb0):
            for blk in range(cw // SBLK):
                base = cb0 + blk * SBLK
                pltpu.sync_copy(src_hbm.at[pl.ds(base, SBLK)], srcs_v)
                pltpu.sync_copy(dst_hbm.at[pl.ds(base, SBLK)], dsts_v)
                for b in range(B):
                    pltpu.async_copy(h_hbm.at[srcs_v.at[b]], rows[b], gs[b])
                lax.fori_loop(0, tbh - 1, body, 0)
                step(tbh - 1, refill=False)

        pltpu.sync_copy(z_hbm.at[pl.ds(0, KG)], rows[0])
        for r in range(rows_per_tile // KG):
            pltpu.sync_copy(rows[0], acc_sh.at[pl.ds(row0 + r * KG, KG)])
        plsc.subcore_barrier()

        @pl.when(c == SLOW_C)
        def _():
            run(cw_s, s * cw_s)

        @pl.when(c != SLOW_C)
        def _():
            run(cw_f, NS * cw_s + s * cw_f)

        plsc.subcore_barrier()
        pltpu.sync_copy(acc_sh.at[pl.ds(row0, rows_per_tile)],
                        out_hbm.at[c, pl.ds(row0, rows_per_tile)])

    return agg(h, src2, dst2, zeros2d)


def _sc_count(dst2, zeros2d, ones2d, npad):
    n_chunks_total, _ = dst2.shape
    D = zeros2d.shape[1]
    chunks_per_w = n_chunks_total // NW
    rows_per_tile = npad // NS
    SB = 8
    ng = chunks_per_w // SB

    mesh = plsc.VectorSubcoreMesh(core_axis_name="c", subcore_axis_name="s")

    @functools.partial(
        pl.kernel,
        out_type=jax.ShapeDtypeStruct((NC, npad, D), jnp.float32),
        mesh=mesh,
        scratch_types=[
            pltpu.VMEM((chunks_per_w, K), jnp.int32),
            pltpu.VMEM((K, D), jnp.float32),
            pltpu.VMEM_SHARED((npad, D), jnp.float32),
        ] + [pltpu.SemaphoreType.DMA] * SB,
    )
    def cnt(dst_hbm, z_hbm, ones_hbm, out_hbm, dsts_v, ones_v, acc_sh, *sems):
        c = lax.axis_index("c")
        s = lax.axis_index("s")
        row0 = s * rows_per_tile
        wid = c * NS + s
        cb0 = wid * chunks_per_w
        pltpu.sync_copy(dst_hbm.at[pl.ds(cb0, chunks_per_w)], dsts_v)
        pltpu.sync_copy(z_hbm.at[pl.ds(0, K)], ones_v)
        for r in range(rows_per_tile // K):
            pltpu.sync_copy(ones_v, acc_sh.at[pl.ds(row0 + r * K, K)])
        pltpu.sync_copy(ones_hbm.at[pl.ds(0, K)], ones_v)
        plsc.subcore_barrier()

        for b in range(SB):
            pltpu.async_copy(ones_v, acc_sh.at[dsts_v.at[b]], sems[b],
                             add=True)

        def body(t, carry):
            jb = t * SB
            for b in range(SB):
                pltpu.make_async_copy(ones_v, acc_sh.at[dsts_v.at[jb - SB + b]],
                                      sems[b]).wait()
                pltpu.async_copy(ones_v, acc_sh.at[dsts_v.at[jb + b]],
                                 sems[b], add=True)
            return carry

        lax.fori_loop(1, ng, body, 0)
        for b in range(SB):
            pltpu.make_async_copy(ones_v,
                                  acc_sh.at[dsts_v.at[(ng - 1) * SB + b]],
                                  sems[b]).wait()
        plsc.subcore_barrier()
        pltpu.sync_copy(acc_sh.at[pl.ds(row0, rows_per_tile)],
                        out_hbm.at[c, pl.ds(row0, rows_per_tile)])

    return cnt(dst2, zeros2d, ones2d)


def _tc_combine(cntp, part, h, Wl, Wr, bl2, relu):
    npad, D = h.shape
    R = 1280
    grid = npad // R

    def body(cnt_ref, part_ref, h_ref, wl_ref, wr_ref, bl_ref, out_ref):
        cb = cnt_ref[0] + cnt_ref[1]
        inv = 1.0 / jnp.maximum(cb, 1.0)
        mean = (part_ref[0] + part_ref[1]) * inv
        acc = lax.dot_general(mean, wl_ref[...], (((1,), (1,)), ((), ())),
                              preferred_element_type=jnp.float32)
        acc = acc + lax.dot_general(h_ref[...], wr_ref[...],
                                    (((1,), (1,)), ((), ())),
                                    preferred_element_type=jnp.float32)
        acc = acc + bl_ref[...]
        if relu:
            acc = jnp.maximum(acc, 0.0)
        out_ref[...] = acc

    return pl.pallas_call(
        body,
        grid=(grid,),
        in_specs=[
            pl.BlockSpec((NC, R, D), lambda i: (0, i, 0)),
            pl.BlockSpec((NC, R, D), lambda i: (0, i, 0)),
            pl.BlockSpec((R, D), lambda i: (i, 0)),
            pl.BlockSpec((D, D), lambda i: (0, 0)),
            pl.BlockSpec((D, D), lambda i: (0, 0)),
            pl.BlockSpec((1, D), lambda i: (0, 0)),
        ],
        out_specs=pl.BlockSpec((R, D), lambda i: (i, 0)),
        out_shape=jax.ShapeDtypeStruct((npad, D), jnp.float32),
    )(cntp, part, h, Wl, Wr, bl2)


def kernel(x, edge_index, Wl0, bl0, Wr0, Wl1, bl1, Wr1, Wl2, bl2, Wr2):
    N, D = x.shape
    E = edge_index.shape[1]
    npad = ((N + 1279) // 1280) * 1280
    echunk = NW * K * 8
    epad = ((E + echunk - 1) // echunk) * echunk

    src = edge_index[0]
    dst = edge_index[1]
    pad = epad - E
    srcp = jnp.concatenate([src, jnp.zeros((pad,), jnp.int32)])
    dstp = jnp.concatenate([dst, jnp.full((pad,), N, jnp.int32)])
    src4 = srcp.reshape(-1, KG)
    dst4 = dstp.reshape(-1, KG)
    dst2 = dstp.reshape(-1, K)

    zeros_kd = jnp.zeros((K, D), jnp.float32)
    ones_kd = jnp.ones((K, D), jnp.float32)
    xp = jnp.concatenate([x, jnp.zeros((npad - N, D), jnp.float32)], axis=0)

    cntp = _sc_count(dst2, zeros_kd, ones_kd, npad)

    n_chunks_total = epad // KG
    tot_per_ns = n_chunks_total // NS
    cw_s = max(SBLK,
               int(round(n_chunks_total * 0.5 / (NS * SBLK))) * SBLK)
    cw_f = tot_per_ns - cw_s

    h = xp
    for Wl, bl, Wr, relu in ((Wl0, bl0, Wr0, True),
                             (Wl1, bl1, Wr1, True),
                             (Wl2, bl2, Wr2, False)):
        part = _sc_aggregate(h, src4, dst4, zeros_kd, npad, cw_s, cw_f)
        h = _tc_combine(cntp, part, h, Wl, Wr, bl.reshape(1, D), relu)
    return h[:N]


# speedup vs baseline: 1.0812x; 1.0055x over previous
import functools

import jax
import jax.numpy as jnp
from jax import lax
from jax.experimental import pallas as pl
from jax.experimental.pallas import tpu as pltpu
from jax.experimental.pallas import tpu_sc as plsc

NC = 2
NS = 16
NW = NC * NS
K = 128
KG = 64
B = 4


SLOW_C = 1
SBLK = 40


def _sc_aggregate(h, src2, dst2, zeros2d, npad, cw_s, cw_f):
    n_chunks_total, _ = src2.shape
    D = h.shape[1]
    rows_per_tile = npad // NS
    tbh = SBLK // B

    mesh = plsc.VectorSubcoreMesh(core_axis_name="c", subcore_axis_name="s")

    @functools.partial(
        pl.kernel,
        out_type=jax.ShapeDtypeStruct((NC, npad, D), jnp.float32),
        mesh=mesh,
        scratch_types=[
            pltpu.VMEM((SBLK, KG), jnp.int32),
            pltpu.VMEM((SBLK, KG), jnp.int32),
        ] + [pltpu.VMEM((KG, D), jnp.float32)] * B + [
            pltpu.VMEM_SHARED((npad, D), jnp.float32),
        ] + [pltpu.SemaphoreType.DMA] * (2 * B),
    )
    def agg(h_hbm, src_hbm, dst_hbm, z_hbm, out_hbm,
            srcs_v, dsts_v, *rest):
        rows, acc_sh, sems = rest[:B], rest[B], rest[B + 1:]
        gs, ss = sems[:B], sems[B:]
        c = lax.axis_index("c")
        s = lax.axis_index("s")
        row0 = s * rows_per_tile

        def gwait(b, j):
            pltpu.make_async_copy(h_hbm.at[srcs_v.at[j]],
                                  rows[b], gs[b]).wait()

        def swait(b, j):
            pltpu.make_async_copy(rows[b],
                                  acc_sh.at[dsts_v.at[j]], ss[b]).wait()

        def step(t, refill):
            jb = t * B
            for b in range(B):
                gwait(b, jb + b)
                pltpu.async_copy(rows[b], acc_sh.at[dsts_v.at[jb + b]],
                                 ss[b], add=True)
            for b in range(B):
                swait(b, jb + b)
                if refill:
                    pltpu.async_copy(h_hbm.at[srcs_v.at[jb + B + b]],
                                     rows[b], gs[b])

        def body(t, carry):
            step(t, refill=True)
            return carry

        def run(cw, cb0):
            for blk in range(cw // SBLK):
                base = cb0 + blk * SBLK
                pltpu.sync_copy(src_hbm.at[pl.ds(base, SBLK)], srcs_v)
                pltpu.sync_copy(dst_hbm.at[pl.ds(base, SBLK)], dsts_v)
                for b in range(B):
                    pltpu.async_copy(h_hbm.at[srcs_v.at[b]], rows[b], gs[b])
                lax.fori_loop(0, tbh - 1, body, 0)
                step(tbh - 1, refill=False)

        pltpu.sync_copy(z_hbm.at[pl.ds(0, KG)], rows[0])
        for r in range(rows_per_tile // KG):
            pltpu.sync_copy(rows[0], acc_sh.at[pl.ds(row0 + r * KG, KG)])
        plsc.subcore_barrier()

        @pl.when(c == SLOW_C)
        def _():
            run(cw_s, s * cw_s)

        @pl.when(c != SLOW_C)
        def _():
            run(cw_f, NS * cw_s + s * cw_f)

        plsc.subcore_barrier()
        pltpu.sync_copy(acc_sh.at[pl.ds(row0, rows_per_tile)],
                        out_hbm.at[c, pl.ds(row0, rows_per_tile)])

    return agg(h, src2, dst2, zeros2d)


def _sc_count(dst2, zeros2d, ones2d, npad):
    n_chunks_total, _ = dst2.shape
    D = zeros2d.shape[1]
    chunks_per_w = n_chunks_total // NW
    rows_per_tile = npad // NS
    SB = 8
    ng = chunks_per_w // SB

    mesh = plsc.VectorSubcoreMesh(core_axis_name="c", subcore_axis_name="s")

    @functools.partial(
        pl.kernel,
        out_type=jax.ShapeDtypeStruct((NC, npad, D), jnp.float32),
        mesh=mesh,
        scratch_types=[
            pltpu.VMEM((chunks_per_w, K), jnp.int32),
            pltpu.VMEM((K, D), jnp.float32),
            pltpu.VMEM_SHARED((npad, D), jnp.float32),
        ] + [pltpu.SemaphoreType.DMA] * SB,
    )
    def cnt(dst_hbm, z_hbm, ones_hbm, out_hbm, dsts_v, ones_v, acc_sh, *sems):
        c = lax.axis_index("c")
        s = lax.axis_index("s")
        row0 = s * rows_per_tile
        wid = c * NS + s
        cb0 = wid * chunks_per_w
        pltpu.sync_copy(dst_hbm.at[pl.ds(cb0, chunks_per_w)], dsts_v)
        pltpu.sync_copy(z_hbm.at[pl.ds(0, K)], ones_v)
        for r in range(rows_per_tile // K):
            pltpu.sync_copy(ones_v, acc_sh.at[pl.ds(row0 + r * K, K)])
        pltpu.sync_copy(ones_hbm.at[pl.ds(0, K)], ones_v)
        plsc.subcore_barrier()

        for b in range(SB):
            pltpu.async_copy(ones_v, acc_sh.at[dsts_v.at[b]], sems[b],
                             add=True)

        def body(t, carry):
            jb = t * SB
            for b in range(SB):
                pltpu.make_async_copy(ones_v, acc_sh.at[dsts_v.at[jb - SB + b]],
                                      sems[b]).wait()
                pltpu.async_copy(ones_v, acc_sh.at[dsts_v.at[jb + b]],
                                 sems[b], add=True)
            return carry

        lax.fori_loop(1, ng, body, 0)
        for b in range(SB):
            pltpu.make_async_copy(ones_v,
                                  acc_sh.at[dsts_v.at[(ng - 1) * SB + b]],
                                  sems[b]).wait()
        plsc.subcore_barrier()
        pltpu.sync_copy(acc_sh.at[pl.ds(row0, rows_per_tile)],
                        out_hbm.at[c, pl.ds(row0, rows_per_tile)])

    return cnt(dst2, zeros2d, ones2d)


def _tc_combine(cntp, part, h, Wl, Wr, bl2, relu):
    npad, D = h.shape
    R = 1280
    grid = npad // R

    def body(cnt_ref, part_ref, h_ref, wl_ref, wr_ref, bl_ref, out_ref):
        cb = cnt_ref[0] + cnt_ref[1]
        inv = 1.0 / jnp.maximum(cb, 1.0)
        mean = (part_ref[0] + part_ref[1]) * inv
        acc = lax.dot_general(mean, wl_ref[...], (((1,), (1,)), ((), ())),
                              preferred_element_type=jnp.float32)
        acc = acc + lax.dot_general(h_ref[...], wr_ref[...],
                                    (((1,), (1,)), ((), ())),
                                    preferred_element_type=jnp.float32)
        acc = acc + bl_ref[...]
        if relu:
            acc = jnp.maximum(acc, 0.0)
        out_ref[...] = acc

    return pl.pallas_call(
        body,
        grid=(grid,),
        in_specs=[
            pl.BlockSpec((NC, R, D), lambda i: (0, i, 0)),
            pl.BlockSpec((NC, R, D), lambda i: (0, i, 0)),
            pl.BlockSpec((R, D), lambda i: (i, 0)),
            pl.BlockSpec((D, D), lambda i: (0, 0)),
            pl.BlockSpec((D, D), lambda i: (0, 0)),
            pl.BlockSpec((1, D), lambda i: (0, 0)),
        ],
        out_specs=pl.BlockSpec((R, D), lambda i: (i, 0)),
        out_shape=jax.ShapeDtypeStruct((npad, D), jnp.float32),
    )(cntp, part, h, Wl, Wr, bl2)


def kernel(x, edge_index, Wl0, bl0, Wr0, Wl1, bl1, Wr1, Wl2, bl2, Wr2):
    N, D = x.shape
    E = edge_index.shape[1]
    npad = ((N + 1279) // 1280) * 1280
    echunk = NW * K * 8
    epad = ((E + echunk - 1) // echunk) * echunk

    src = edge_index[0]
    dst = edge_index[1]
    pad = epad - E
    srcp = jnp.concatenate([src, jnp.zeros((pad,), jnp.int32)])
    dstp = jnp.concatenate([dst, jnp.full((pad,), N, jnp.int32)])
    src4 = srcp.reshape(-1, KG)
    dst4 = dstp.reshape(-1, KG)
    dst2 = dstp.reshape(-1, K)

    zeros_kd = jnp.zeros((K, D), jnp.float32)
    ones_kd = jnp.ones((K, D), jnp.float32)
    xp = jnp.concatenate([x, jnp.zeros((npad - N, D), jnp.float32)], axis=0)

    cntp = _sc_count(dst2, zeros_kd, ones_kd, npad)

    n_chunks_total = epad // KG
    tot_per_ns = n_chunks_total // NS
    cw_s = max(SBLK,
               int(round(n_chunks_total * 0.5 / (NS * SBLK))) * SBLK)
    cw_f = tot_per_ns - cw_s

    h = xp
    for Wl, bl, Wr, relu in ((Wl0, bl0, Wr0, True),
                             (Wl1, bl1, Wr1, True),
                             (Wl2, bl2, Wr2, False)):
        part = _sc_aggregate(h, src4, dst4, zeros_kd, npad, cw_s, cw_f)
        h = _tc_combine(cntp, part, h, Wl, Wr, bl.reshape(1, D), relu)
    return h[:N]


# speedup vs baseline: 1.1442x; 1.0582x over previous
import functools

import jax
import jax.numpy as jnp
from jax import lax
from jax.experimental import pallas as pl
from jax.experimental.pallas import tpu as pltpu
from jax.experimental.pallas import tpu_sc as plsc

NC = 2
NS = 16
NW = NC * NS
K = 128
KG = 64
B = 4


SBLK = 40


def _sc_aggregate(h, src2, dst2, zeros2d, npad):
    n_chunks_total, _ = src2.shape
    D = h.shape[1]
    cw = n_chunks_total // NW
    rows_per_tile = npad // NS
    tbh = SBLK // B

    mesh = plsc.VectorSubcoreMesh(core_axis_name="c", subcore_axis_name="s")

    @functools.partial(
        pl.kernel,
        out_type=jax.ShapeDtypeStruct((NC, npad, D), jnp.float32),
        mesh=mesh,
        scratch_types=[
            pltpu.VMEM((SBLK, KG), jnp.int32),
            pltpu.VMEM((SBLK, KG), jnp.int32),
        ] + [pltpu.VMEM((KG, D), jnp.float32)] * B + [
            pltpu.VMEM_SHARED((npad, D), jnp.float32),
        ] + [pltpu.SemaphoreType.DMA] * (2 * B),
    )
    def agg(h_hbm, src_hbm, dst_hbm, z_hbm, out_hbm,
            srcs_v, dsts_v, *rest):
        rows, acc_sh, sems = rest[:B], rest[B], rest[B + 1:]
        gs, ss = sems[:B], sems[B:]
        c = lax.axis_index("c")
        s = lax.axis_index("s")
        row0 = s * rows_per_tile

        def gwait(b, j):
            pltpu.make_async_copy(h_hbm.at[srcs_v.at[j]],
                                  rows[b], gs[b]).wait()

        def swait(b, j):
            pltpu.make_async_copy(rows[b],
                                  acc_sh.at[dsts_v.at[j]], ss[b]).wait()

        def step(t, refill):
            jb = t * B
            for b in range(B):
                gwait(b, jb + b)
                pltpu.async_copy(rows[b], acc_sh.at[dsts_v.at[jb + b]],
                                 ss[b], add=True)
            for b in range(B):
                swait(b, jb + b)
                if refill:
                    pltpu.async_copy(h_hbm.at[srcs_v.at[jb + B + b]],
                                     rows[b], gs[b])

        def body(t, carry):
            step(t, refill=True)
            return carry

        wid = c * NS + s
        cb0 = wid * cw
        for blk in range(cw // SBLK):
            base = cb0 + blk * SBLK
            pltpu.sync_copy(src_hbm.at[pl.ds(base, SBLK)], srcs_v)
            pltpu.sync_copy(dst_hbm.at[pl.ds(base, SBLK)], dsts_v)
            if blk == 0:
                pltpu.sync_copy(z_hbm.at[pl.ds(0, KG)], rows[0])
                for r in range(rows_per_tile // KG):
                    pltpu.sync_copy(rows[0],
                                    acc_sh.at[pl.ds(row0 + r * KG, KG)])
            for b in range(B):
                pltpu.async_copy(h_hbm.at[srcs_v.at[b]], rows[b], gs[b])
            if blk == 0:
                plsc.subcore_barrier()
            lax.fori_loop(0, tbh - 1, body, 0)
            step(tbh - 1, refill=False)

        plsc.subcore_barrier()
        pltpu.sync_copy(acc_sh.at[pl.ds(row0, rows_per_tile)],
                        out_hbm.at[c, pl.ds(row0, rows_per_tile)])

    return agg(h, src2, dst2, zeros2d)


def _sc_count(dst2, zeros2d, ones2d, npad):
    n_chunks_total, _ = dst2.shape
    D = zeros2d.shape[1]
    chunks_per_w = n_chunks_total // NW
    rows_per_tile = npad // NS
    SB = 8
    ng = chunks_per_w // SB

    mesh = plsc.VectorSubcoreMesh(core_axis_name="c", subcore_axis_name="s")

    @functools.partial(
        pl.kernel,
        out_type=jax.ShapeDtypeStruct((NC, npad, D), jnp.float32),
        mesh=mesh,
        scratch_types=[
            pltpu.VMEM((chunks_per_w, K), jnp.int32),
            pltpu.VMEM((K, D), jnp.float32),
            pltpu.VMEM_SHARED((npad, D), jnp.float32),
        ] + [pltpu.SemaphoreType.DMA] * SB,
    )
    def cnt(dst_hbm, z_hbm, ones_hbm, out_hbm, dsts_v, ones_v, acc_sh, *sems):
        c = lax.axis_index("c")
        s = lax.axis_index("s")
        row0 = s * rows_per_tile
        wid = c * NS + s
        cb0 = wid * chunks_per_w
        pltpu.sync_copy(dst_hbm.at[pl.ds(cb0, chunks_per_w)], dsts_v)
        pltpu.sync_copy(z_hbm.at[pl.ds(0, K)], ones_v)
        for r in range(rows_per_tile // K):
            pltpu.sync_copy(ones_v, acc_sh.at[pl.ds(row0 + r * K, K)])
        pltpu.sync_copy(ones_hbm.at[pl.ds(0, K)], ones_v)
        plsc.subcore_barrier()

        for b in range(SB):
            pltpu.async_copy(ones_v, acc_sh.at[dsts_v.at[b]], sems[b],
                             add=True)

        def body(t, carry):
            jb = t * SB
            for b in range(SB):
                pltpu.make_async_copy(ones_v, acc_sh.at[dsts_v.at[jb - SB + b]],
                                      sems[b]).wait()
                pltpu.async_copy(ones_v, acc_sh.at[dsts_v.at[jb + b]],
                                 sems[b], add=True)
            return carry

        lax.fori_loop(1, ng, body, 0)
        for b in range(SB):
            pltpu.make_async_copy(ones_v,
                                  acc_sh.at[dsts_v.at[(ng - 1) * SB + b]],
                                  sems[b]).wait()
        plsc.subcore_barrier()
        pltpu.sync_copy(acc_sh.at[pl.ds(row0, rows_per_tile)],
                        out_hbm.at[c, pl.ds(row0, rows_per_tile)])

    return cnt(dst2, zeros2d, ones2d)


def _tc_combine(cntp, part, h, Wl, Wr, bl2, relu):
    npad, D = h.shape
    R = 1280
    grid = npad // R

    def body(cnt_ref, part_ref, h_ref, wl_ref, wr_ref, bl_ref, out_ref):
        cb = cnt_ref[0] + cnt_ref[1]
        inv = 1.0 / jnp.maximum(cb, 1.0)
        mean = (part_ref[0] + part_ref[1]) * inv
        acc = lax.dot_general(mean, wl_ref[...], (((1,), (1,)), ((), ())),
                              preferred_element_type=jnp.float32)
        acc = acc + lax.dot_general(h_ref[...], wr_ref[...],
                                    (((1,), (1,)), ((), ())),
                                    preferred_element_type=jnp.float32)
        acc = acc + bl_ref[...]
        if relu:
            acc = jnp.maximum(acc, 0.0)
        out_ref[...] = acc

    return pl.pallas_call(
        body,
        grid=(grid,),
        in_specs=[
            pl.BlockSpec((NC, R, D), lambda i: (0, i, 0)),
            pl.BlockSpec((NC, R, D), lambda i: (0, i, 0)),
            pl.BlockSpec((R, D), lambda i: (i, 0)),
            pl.BlockSpec((D, D), lambda i: (0, 0)),
            pl.BlockSpec((D, D), lambda i: (0, 0)),
            pl.BlockSpec((1, D), lambda i: (0, 0)),
        ],
        out_specs=pl.BlockSpec((R, D), lambda i: (i, 0)),
        out_shape=jax.ShapeDtypeStruct((npad, D), jnp.float32),
    )(cntp, part, h, Wl, Wr, bl2)


def kernel(x, edge_index, Wl0, bl0, Wr0, Wl1, bl1, Wr1, Wl2, bl2, Wr2):
    N, D = x.shape
    E = edge_index.shape[1]
    npad = ((N + 1279) // 1280) * 1280
    echunk = NW * K * 8
    epad = ((E + echunk - 1) // echunk) * echunk

    src = edge_index[0]
    dst = edge_index[1]
    pad = epad - E
    srcp = jnp.concatenate([src, jnp.zeros((pad,), jnp.int32)])
    dstp = jnp.concatenate([dst, jnp.full((pad,), N, jnp.int32)])
    src4 = srcp.reshape(-1, KG)
    dst4 = dstp.reshape(-1, KG)
    dst2 = dstp.reshape(-1, K)

    zeros_kd = jnp.zeros((K, D), jnp.float32)
    ones_kd = jnp.ones((K, D), jnp.float32)
    xp = jnp.concatenate([x, jnp.zeros((npad - N, D), jnp.float32)], axis=0)

    cntp = _sc_count(dst2, zeros_kd, ones_kd, npad)

    h = xp
    for Wl, bl, Wr, relu in ((Wl0, bl0, Wr0, True),
                             (Wl1, bl1, Wr1, True),
                             (Wl2, bl2, Wr2, False)):
        part = _sc_aggregate(h, src4, dst4, zeros_kd, npad)
        h = _tc_combine(cntp, part, h, Wl, Wr, bl.reshape(1, D), relu)
    return h[:N]
